# baseline (speedup 1.0000x reference)
import jax
import jax.numpy as jnp
from jax.experimental import pallas as pl

_K = 256
_PART = 16384
_ROWS_PER_BLOCK = 128


def _decode(cand):
    u = jnp.where((cand >> jnp.uint32(31)) == jnp.uint32(1),
                  cand ^ jnp.uint32(0x80000000), ~cand)
    return jax.lax.bitcast_convert_type(u, jnp.float32)


def _decode16(cand):
    u = jnp.where((cand >> jnp.uint32(15)) == jnp.uint32(1),
                  cand ^ jnp.uint32(0x8000),
                  ~cand & jnp.uint32(0xFFFF))
    return jax.lax.bitcast_convert_type(u.astype(jnp.uint16), jnp.bfloat16)


def _split_topk_kernel(x_ref, o_ref):
    x = x_ref[...]
    rows = x.shape[0]
    cols = x.shape[1]
    u = jax.lax.bitcast_convert_type(x, jnp.uint32)
    xb = jax.lax.bitcast_convert_type(
        (u >> jnp.uint32(16)).astype(jnp.uint16), jnp.bfloat16)
    one_b = jnp.full(xb.shape, 1, jnp.bfloat16)
    zero_b = jnp.zeros(xb.shape, jnp.bfloat16)
    p = jnp.zeros((rows, 1), jnp.uint32)
    for b in range(15, -1, -1):
        cand = p | jnp.uint32(1 << b)
        m = jnp.where(xb >= _decode16(cand), one_b, zero_b)
        partial = m[:, :256]
        for i in range(1, cols // 256):
            partial = partial + m[:, 256 * i:256 * (i + 1)]
        cnt = jnp.sum(partial.astype(jnp.float32), axis=1, keepdims=True)
        p = jnp.where(cnt >= _K, cand, p)
    p_bf = _decode16(p)
    m = jnp.where(xb > p_bf, one_b, zero_b)
    partial = m[:, :256]
    for i in range(1, cols // 256):
        partial = partial + m[:, 256 * i:256 * (i + 1)]
    c_gt = jnp.sum(partial.astype(jnp.float32), axis=1, keepdims=True)
    band = xb == p_bf
    xl = jax.lax.bitcast_convert_type(
        (u & jnp.uint32(0xFFFF)).astype(jnp.uint16), jnp.int16)
    xl = xl ^ jnp.int16(-0x8000)
    xl = jnp.where(band, xl, jnp.int16(-0x8000))
    t = p << jnp.uint32(16)
    for b in range(15, -1, -1):
        cand = t | jnp.uint32(1 << b)
        cl = ((cand & jnp.uint32(0xFFFF)).astype(jnp.int32)
              - 0x8000).astype(jnp.int16)
        m = jnp.where(xl >= cl, one_b, zero_b)
        partial = m[:, :256]
        for i in range(1, cols // 256):
            partial = partial + m[:, 256 * i:256 * (i + 1)]
        cnt = c_gt + jnp.sum(partial.astype(jnp.float32), axis=1,
                             keepdims=True)
        t = jnp.where(cnt >= _K, cand, t)
    o_ref[...] = jnp.where(x >= _decode(t), jnp.maximum(x, 0.0), 0.0)


@jax.jit
def kernel(x):
    m, n = x.shape
    grid = (m // _ROWS_PER_BLOCK, n // _PART)
    return pl.pallas_call(
        _split_topk_kernel,
        grid=grid,
        in_specs=[pl.BlockSpec((_ROWS_PER_BLOCK, _PART), lambda i, j: (i, j))],
        out_specs=pl.BlockSpec((_ROWS_PER_BLOCK, _PART), lambda i, j: (i, j)),
        out_shape=jax.ShapeDtypeStruct((m, n), x.dtype),
    )(x)


# speedup vs baseline: 1.0209x; 1.0209x over previous
import jax
import jax.numpy as jnp
from jax.experimental import pallas as pl

_K = 256
_PART = 16384
_ROWS_PER_BLOCK = 64


def _decode(cand):
    u = jnp.where((cand >> jnp.uint32(31)) == jnp.uint32(1),
                  cand ^ jnp.uint32(0x80000000), ~cand)
    return jax.lax.bitcast_convert_type(u, jnp.float32)


def _decode16(cand):
    u = jnp.where((cand >> jnp.uint32(15)) == jnp.uint32(1),
                  cand ^ jnp.uint32(0x8000),
                  ~cand & jnp.uint32(0xFFFF))
    return jax.lax.bitcast_convert_type(u.astype(jnp.uint16), jnp.bfloat16)


def _split_topk_kernel(x_ref, o_ref):
    x = x_ref[...]
    rows = x.shape[0]
    cols = x.shape[1]
    u = jax.lax.bitcast_convert_type(x, jnp.uint32)
    xb = jax.lax.bitcast_convert_type(
        (u >> jnp.uint32(16)).astype(jnp.uint16), jnp.bfloat16)
    one_b = jnp.full(xb.shape, 1, jnp.bfloat16)
    zero_b = jnp.zeros(xb.shape, jnp.bfloat16)
    p = jnp.zeros((rows, 1), jnp.uint32)
    for b in range(15, -1, -1):
        cand = p | jnp.uint32(1 << b)
        m = jnp.where(xb >= _decode16(cand), one_b, zero_b)
        partial = m[:, :256]
        for i in range(1, cols // 256):
            partial = partial + m[:, 256 * i:256 * (i + 1)]
        cnt = jnp.sum(partial.astype(jnp.float32), axis=1, keepdims=True)
        p = jnp.where(cnt >= _K, cand, p)
    p_bf = _decode16(p)
    m = jnp.where(xb > p_bf, one_b, zero_b)
    partial = m[:, :256]
    for i in range(1, cols // 256):
        partial = partial + m[:, 256 * i:256 * (i + 1)]
    c_gt = jnp.sum(partial.astype(jnp.float32), axis=1, keepdims=True)
    band = xb == p_bf
    xl = jax.lax.bitcast_convert_type(
        (u & jnp.uint32(0xFFFF)).astype(jnp.uint16), jnp.int16)
    xl = xl ^ jnp.int16(-0x8000)
    xl = jnp.where(band, xl, jnp.int16(-0x8000))
    t = p << jnp.uint32(16)
    for b in range(15, -1, -1):
        cand = t | jnp.uint32(1 << b)
        cl = ((cand & jnp.uint32(0xFFFF)).astype(jnp.int32)
              - 0x8000).astype(jnp.int16)
        m = jnp.where(xl >= cl, one_b, zero_b)
        partial = m[:, :256]
        for i in range(1, cols // 256):
            partial = partial + m[:, 256 * i:256 * (i + 1)]
        cnt = c_gt + jnp.sum(partial.astype(jnp.float32), axis=1,
                             keepdims=True)
        t = jnp.where(cnt >= _K, cand, t)
    o_ref[...] = jnp.where(x >= _decode(t), jnp.maximum(x, 0.0), 0.0)


@jax.jit
def kernel(x):
    m, n = x.shape
    grid = (m // _ROWS_PER_BLOCK, n // _PART)
    return pl.pallas_call(
        _split_topk_kernel,
        grid=grid,
        in_specs=[pl.BlockSpec((_ROWS_PER_BLOCK, _PART), lambda i, j: (i, j))],
        out_specs=pl.BlockSpec((_ROWS_PER_BLOCK, _PART), lambda i, j: (i, j)),
        out_shape=jax.ShapeDtypeStruct((m, n), x.dtype),
    )(x)
